# Initial kernel scaffold; baseline (speedup 1.0000x reference)
import jax
import jax.numpy as jnp
from jax.experimental import pallas as pl


def kernel(features, support, segment_ids, weights):
    raise NotImplementedError("write your pallas kernel here")


# speedup vs baseline: 7.7591x; 7.7591x over previous
import jax
import jax.numpy as jnp
from jax.experimental import pallas as pl
from jax.experimental.pallas import tpu as pltpu

N_TOK = 16384
D = 512
B = 16
BLK = 1024
NB = N_TOK // BLK


def _body(feat_ref, ids_ref, w_ref, out_ref, acc_ref, cnt_ref, g_ref):
    p = pl.program_id(0)
    i = pl.program_id(1)
    f = feat_ref[:]
    ids = ids_ref[:]
    bmin = jnp.min(ids)
    bmax = jnp.max(ids)

    @pl.when(jnp.logical_and(p == 0, i == 0))
    def _init0():
        acc_ref[:] = jnp.zeros((B, D), jnp.float32)
        cnt_ref[:] = jnp.zeros((B, 128), jnp.float32)

    @pl.when(p == 0)
    def _phase0():
        def seg_body(b, _):
            mask = (ids == b).astype(jnp.float32)
            acc_ref[pl.ds(b, 1), :] += jnp.sum(f * mask, axis=0, keepdims=True)
            cnt_ref[pl.ds(b, 1), :] += jnp.sum(mask)
            return 0
        jax.lax.fori_loop(bmin, bmax + 1, seg_body, 0)

        @pl.when(i == NB - 1)
        def _make_g():
            cnt = jnp.maximum(cnt_ref[:, 0:1], 1.0)
            mean = acc_ref[:] / cnt
            g_ref[:] = jnp.maximum(
                jax.lax.dot_general(mean, w_ref[:], (((1,), (1,)), ((), ())),
                                    preferred_element_type=jnp.float32),
                0.0)

    @pl.when(p == 1)
    def _phase1():
        @pl.when(i == 0)
        def _init1():
            out_ref[:] = jnp.zeros((B, D), jnp.float32)

        def seg_body(b, _):
            mask = (ids == b).astype(jnp.float32)
            gb = g_ref[pl.ds(b, 1), :]
            att = jnp.sum(f * gb, axis=1, keepdims=True)
            att = jnp.maximum(att, 0.0) * mask
            out_ref[pl.ds(b, 1), :] += jnp.sum(f * att, axis=0, keepdims=True)
            return 0
        jax.lax.fori_loop(bmin, bmax + 1, seg_body, 0)


def kernel(features, support, segment_ids, weights):
    del support
    ids2 = segment_ids.reshape(N_TOK, 1)
    return pl.pallas_call(
        _body,
        grid=(2, NB),
        in_specs=[
            pl.BlockSpec((BLK, D), lambda p, i: (i, 0)),
            pl.BlockSpec((BLK, 1), lambda p, i: (i, 0)),
            pl.BlockSpec((D, D), lambda p, i: (0, 0)),
        ],
        out_specs=pl.BlockSpec((B, D), lambda p, i: (0, 0)),
        out_shape=jax.ShapeDtypeStruct((B, D), jnp.float32),
        scratch_shapes=[
            pltpu.VMEM((B, D), jnp.float32),
            pltpu.VMEM((B, 128), jnp.float32),
            pltpu.VMEM((B, D), jnp.float32),
        ],
        compiler_params=pltpu.CompilerParams(
            dimension_semantics=("arbitrary", "arbitrary")),
    )(features, ids2, weights)
